# baseline (speedup 1.0000x reference)
import functools

import jax
import jax.numpy as jnp
from jax import lax
from jax.experimental import pallas as pl
from jax.experimental.pallas import tpu as pltpu
from jax.experimental.pallas import tpu_sc as plsc

B = 16384
V = 100000
D = 32
NF = 4
ROUTING_DIM = 128

_info = plsc.get_sparse_core_info()
_NC, _NS = _info.num_cores, _info.num_subcores
_NW = _NC * _NS
_IC = 4096


_H0 = 50048
_H1 = V - _H0


def _sc_gather_body(i0, i1, i2, i3, t0, t1, t2, t3, out_hbm,
                    buf_a, buf_b, idx_sh, idx_v, out_v, sem_a, sem_b, sem_o):
    sid = lax.axis_index("s")
    wid = sid * _NC + lax.axis_index("c")
    idx_refs = (i0, i1, i2, i3)
    tab_refs = (t0, t1, t2, t3)

    for f in range(NF):
        @pl.when(sid == f)
        def _():
            pltpu.sync_copy(idx_refs[f], idx_sh.at[f])

    cp_a = pltpu.async_copy(tab_refs[0].at[wid, pl.ds(0, _H0)], buf_a, sem_a)
    plsc.subcore_barrier()

    def sweep(f, is_merge, buf):
        def idx_chunk(c, _):
            pltpu.sync_copy(idx_sh.at[f, pl.ds(c * _IC, _IC)], idx_v)

            @plsc.parallel_loop(0, _IC // 16, unroll=8)
            def gather16(j):
                pos = c * _IC + j * 16
                iv = idx_v[pl.ds(j * 16, 16)]
                if is_merge:
                    m = iv >= _H0
                    vals = plsc.load_gather(buf, [iv - _H0], mask=m)
                    prev = out_v[pl.ds(pos, 16)]
                    out_v[pl.ds(pos, 16)] = jnp.where(m, vals, prev)
                else:
                    m = iv < _H0
                    vals = plsc.load_gather(buf, [iv], mask=m)
                    out_v[pl.ds(pos, 16)] = jnp.where(m, vals, 0.0)

            return ()

        lax.fori_loop(0, B // _IC, idx_chunk, ())

    cp_o = None
    for f in range(NF):
        cp_b = pltpu.async_copy(tab_refs[f].at[wid, pl.ds(_H0, _H1)],
                                buf_b, sem_b)
        cp_a.wait()
        if cp_o is not None:
            cp_o.wait()
        sweep(f, False, buf_a)
        cp_b.wait()
        if f < NF - 1:
            cp_a = pltpu.async_copy(tab_refs[f + 1].at[wid, pl.ds(0, _H0)],
                                    buf_a, sem_a)
        sweep(f, True, buf_b)
        cp_o = pltpu.async_copy(out_v, out_hbm.at[f * _NW + wid], sem_o)
    cp_o.wait()


@jax.jit
def _sc_gather(i0, i1, i2, i3, t0, t1, t2, t3):
    mesh = plsc.VectorSubcoreMesh(core_axis_name="c", subcore_axis_name="s")
    return pl.kernel(
        _sc_gather_body,
        mesh=mesh,
        compiler_params=pltpu.CompilerParams(needs_layout_passes=False),
        out_type=jax.ShapeDtypeStruct((NF * D, B), jnp.float32),
        scratch_types=[
            pltpu.VMEM((_H0,), jnp.float32),
            pltpu.VMEM((_H1,), jnp.float32),
            pltpu.VMEM_SHARED((NF, B), jnp.int32),
            pltpu.VMEM((_IC,), jnp.int32),
            pltpu.VMEM((B,), jnp.float32),
            pltpu.SemaphoreType.DMA,
            pltpu.SemaphoreType.DMA,
            pltpu.SemaphoreType.DMA,
        ],
    )(i0, i1, i2, i3, t0, t1, t2, t3)


_BB = 4096


def _proj_body(e_ref, w_ref, b_ref, o_ref):
    o_ref[...] = jax.lax.dot_general(
        e_ref[...].astype(jnp.bfloat16), w_ref[...].astype(jnp.bfloat16),
        (((0,), (0,)), ((), ())),
        preferred_element_type=jnp.float32) + b_ref[...]


@jax.jit
def _tc_project(embsT, W, b2d):
    return pl.pallas_call(
        _proj_body,
        grid=(B // _BB,),
        in_specs=[
            pl.BlockSpec((NF * D, _BB), lambda i: (0, i)),
            pl.BlockSpec((NF * D, ROUTING_DIM), lambda i: (0, 0)),
            pl.BlockSpec((1, ROUTING_DIM), lambda i: (0, 0)),
        ],
        out_specs=pl.BlockSpec((_BB, ROUTING_DIM), lambda i: (i, 0)),
        out_shape=jax.ShapeDtypeStruct((B, ROUTING_DIM), jnp.float32),
    )(embsT, W, b2d)


def kernel(user_id, item_id, category_id, context_id,
           table_0, table_1, table_2, table_3, W, b):
    embsT = _sc_gather(user_id, item_id, category_id, context_id,
                       table_0.T, table_1.T, table_2.T, table_3.T)
    return _tc_project(embsT, W, b.reshape(1, ROUTING_DIM))


# speedup vs baseline: 1.0233x; 1.0233x over previous
import functools

import jax
import jax.numpy as jnp
from jax import lax
from jax.experimental import pallas as pl
from jax.experimental.pallas import tpu as pltpu
from jax.experimental.pallas import tpu_sc as plsc

B = 16384
V = 100000
D = 32
NF = 4
ROUTING_DIM = 128

_info = plsc.get_sparse_core_info()
_NC, _NS = _info.num_cores, _info.num_subcores
_NW = _NC * _NS
_IC = 4096


_H0 = 50048
_H1 = V - _H0


def _sc_gather_body(i0, i1, i2, i3, t0, t1, t2, t3, out_hbm,
                    buf_a, buf_b, idx_sh, idx_v, out_v, sem_a, sem_b, sem_o):
    sid = lax.axis_index("s")
    wid = sid * _NC + lax.axis_index("c")
    idx_refs = (i0, i1, i2, i3)
    tab_refs = (t0, t1, t2, t3)

    for f in range(NF):
        @pl.when(sid == f)
        def _():
            pltpu.sync_copy(idx_refs[f], idx_sh.at[f])

    cp_a = pltpu.async_copy(tab_refs[0].at[wid, pl.ds(0, _H0)], buf_a, sem_a)
    plsc.subcore_barrier()

    def sweep(f, is_merge, buf):
        def idx_chunk(c, _):
            pltpu.sync_copy(idx_sh.at[f, pl.ds(c * _IC, _IC)], idx_v)

            @plsc.parallel_loop(0, _IC // 16, unroll=8)
            def gather16(j):
                pos = c * _IC + j * 16
                iv = idx_v[pl.ds(j * 16, 16)]
                if is_merge:
                    m = iv >= _H0
                    vals = plsc.load_gather(buf, [iv - _H0], mask=m)
                    prev = out_v[pl.ds(pos, 16)]
                    out_v[pl.ds(pos, 16)] = jnp.where(m, vals, prev)
                else:
                    m = iv < _H0
                    vals = plsc.load_gather(buf, [iv], mask=m)
                    out_v[pl.ds(pos, 16)] = jnp.where(m, vals, 0.0)

            return ()

        lax.fori_loop(0, B // _IC, idx_chunk, ())

    cp_o = None
    for f in range(NF):
        cp_b = pltpu.async_copy(tab_refs[f].at[wid, pl.ds(_H0, _H1)],
                                buf_b, sem_b)
        cp_a.wait()
        if cp_o is not None:
            cp_o.wait()
        sweep(f, False, buf_a)
        cp_b.wait()
        if f < NF - 1:
            cp_a = pltpu.async_copy(tab_refs[f + 1].at[wid, pl.ds(0, _H0)],
                                    buf_a, sem_a)
        sweep(f, True, buf_b)
        cp_o = pltpu.async_copy(out_v, out_hbm.at[f * _NW + wid], sem_o)
    cp_o.wait()


@jax.jit
def _sc_gather(i0, i1, i2, i3, t0, t1, t2, t3):
    mesh = plsc.VectorSubcoreMesh(core_axis_name="c", subcore_axis_name="s")
    return pl.kernel(
        _sc_gather_body,
        mesh=mesh,
        compiler_params=pltpu.CompilerParams(needs_layout_passes=False),
        out_type=jax.ShapeDtypeStruct((NF * D, B), jnp.float32),
        scratch_types=[
            pltpu.VMEM((_H0,), jnp.float32),
            pltpu.VMEM((_H1,), jnp.float32),
            pltpu.VMEM_SHARED((NF, B), jnp.int32),
            pltpu.VMEM((_IC,), jnp.int32),
            pltpu.VMEM((B,), jnp.float32),
            pltpu.SemaphoreType.DMA,
            pltpu.SemaphoreType.DMA,
            pltpu.SemaphoreType.DMA,
        ],
    )(i0, i1, i2, i3, t0, t1, t2, t3)


_BB = 8192


def _proj_body(e_ref, w_ref, b_ref, o_ref):
    o_ref[...] = jax.lax.dot_general(
        e_ref[...], w_ref[...], (((0,), (0,)), ((), ())),
        preferred_element_type=jnp.float32) + b_ref[...]


@jax.jit
def _tc_project(embsT, W, b2d):
    return pl.pallas_call(
        _proj_body,
        grid=(B // _BB,),
        in_specs=[
            pl.BlockSpec((NF * D, _BB), lambda i: (0, i)),
            pl.BlockSpec((NF * D, ROUTING_DIM), lambda i: (0, 0)),
            pl.BlockSpec((1, ROUTING_DIM), lambda i: (0, 0)),
        ],
        out_specs=pl.BlockSpec((_BB, ROUTING_DIM), lambda i: (i, 0)),
        out_shape=jax.ShapeDtypeStruct((B, ROUTING_DIM), jnp.float32),
    )(embsT, W, b2d)


def kernel(user_id, item_id, category_id, context_id,
           table_0, table_1, table_2, table_3, W, b):
    embsT = _sc_gather(user_id, item_id, category_id, context_id,
                       table_0.T, table_1.T, table_2.T, table_3.T)
    return _tc_project(embsT, W, b.reshape(1, ROUTING_DIM))
